# Initial kernel scaffold; baseline (speedup 1.0000x reference)
import jax
import jax.numpy as jnp
from jax.experimental import pallas as pl


def kernel(x_paper, x_author, edge_index_pcp, edge_index_awp, edge_index_pwa, Wn_0_pcp, Wr_0_pcp, bn_0_pcp, Wn_0_awp, Wr_0_awp, bn_0_awp, Wn_0_pwa, Wr_0_pwa, bn_0_pwa, Wn_1_pcp, Wr_1_pcp, bn_1_pcp, Wn_1_awp, Wr_1_awp, bn_1_awp, Wn_1_pwa, Wr_1_pwa, bn_1_pwa, W_lin, b_lin):
    raise NotImplementedError("write your pallas kernel here")


# speedup vs baseline: 2.2165x; 2.2165x over previous
import functools

import jax
import jax.numpy as jnp
from jax import lax
from jax.experimental import pallas as pl
from jax.experimental.pallas import tpu as pltpu
from jax.experimental.pallas import tpu_sc as plsc

F32 = jnp.float32

NUM_P = 10000
NUM_A = 10000
NUM_E = 200000
DIM = 128
DIM_OUT = 64

NC = 2
NS = 16
NW = NC * NS
K = 128
NCH = 50
EPW = NCH * K
E_PAD = NW * EPW
N_PAD = 10240
ROWS_PT = N_PAD // NS
PAD_ROW = 10000

_MESH = plsc.VectorSubcoreMesh(
    core_axis_name="c", subcore_axis_name="s", num_cores=NC, num_subcores=NS)


def _make_segsum(num_types):
  out_t = [jax.ShapeDtypeStruct((NC, N_PAD, DIM), F32)] * num_types
  scratch = [
      pltpu.VMEM((NCH, K), jnp.int32),
      pltpu.VMEM((NCH, K), jnp.int32),
      pltpu.VMEM((K, DIM), F32),
      pltpu.VMEM_SHARED((N_PAD, DIM), F32),
      pltpu.SemaphoreType.DMA,
  ]

  @functools.partial(pl.kernel, out_type=out_t, mesh=_MESH,
                     scratch_types=scratch)
  def seg_kernel(*refs):
    xs = refs[:num_types]
    srcs = refs[num_types:2 * num_types]
    dsts = refs[2 * num_types:3 * num_types]
    zeros_hbm = refs[3 * num_types]
    outs = refs[3 * num_types + 1:4 * num_types + 1]
    src_v, dst_v, rows_v, acc, sem = refs[4 * num_types + 1:]
    c = lax.axis_index("c")
    s = lax.axis_index("s")
    wid = c * NS + s
    for t in range(num_types):
      pltpu.sync_copy(zeros_hbm, acc.at[pl.ds(s * ROWS_PT, ROWS_PT)])
      pltpu.sync_copy(srcs[t].at[wid], src_v)
      pltpu.sync_copy(dsts[t].at[wid], dst_v)
      plsc.subcore_barrier()

      def chunk(j, carry, t=t):
        pltpu.async_copy(xs[t].at[src_v.at[j]], rows_v, sem).wait()
        pltpu.sync_copy(rows_v, acc.at[dst_v.at[j]], add=True)
        return carry

      lax.fori_loop(0, NCH, chunk, 0)
      plsc.subcore_barrier()
      pltpu.sync_copy(acc.at[pl.ds(s * ROWS_PT, ROWS_PT)],
                      outs[t].at[c, pl.ds(s * ROWS_PT, ROWS_PT)])

  return seg_kernel


def _make_counts():
  out_t = [jax.ShapeDtypeStruct((NC, N_PAD, DIM), F32)] * 3
  scratch = [
      pltpu.VMEM((NCH, K), jnp.int32),
      pltpu.VMEM((K, DIM), F32),
      pltpu.VMEM_SHARED((N_PAD, DIM), F32),
  ]

  @functools.partial(pl.kernel, out_type=out_t, mesh=_MESH,
                     scratch_types=scratch)
  def cnt_kernel(d0, d1, d2, ones_hbm, zeros_hbm, o0, o1, o2,
                 dst_v, ones_v, acc):
    c = lax.axis_index("c")
    s = lax.axis_index("s")
    wid = c * NS + s
    pltpu.sync_copy(ones_hbm, ones_v)
    for t, (dref, oref) in enumerate(((d0, o0), (d1, o1), (d2, o2))):
      pltpu.sync_copy(zeros_hbm, acc.at[pl.ds(s * ROWS_PT, ROWS_PT)])
      pltpu.sync_copy(dref.at[wid], dst_v)
      plsc.subcore_barrier()

      def chunk(j, carry, t=t):
        pltpu.sync_copy(ones_v, acc.at[dst_v.at[j]], add=True)
        return carry

      lax.fori_loop(0, NCH, chunk, 0)
      plsc.subcore_barrier()
      pltpu.sync_copy(acc.at[pl.ds(s * ROWS_PT, ROWS_PT)],
                      oref.at[c, pl.ds(s * ROWS_PT, ROWS_PT)])

  return cnt_kernel


_SEG3 = _make_segsum(3)
_SEG2 = _make_segsum(2)
_COUNTS = _make_counts()

BLK = 2048


def _acc_spec():
  return pl.BlockSpec((NC, BLK, DIM), lambda i: (0, i, 0))


def _cnt_spec():
  return pl.BlockSpec((NC, BLK, DIM), lambda i: (0, i, 0))


def _x_spec():
  return pl.BlockSpec((BLK, DIM), lambda i: (i, 0))


def _w_spec():
  return pl.BlockSpec((DIM, DIM), lambda i: (0, 0))


def _b_spec():
  return pl.BlockSpec((1, DIM), lambda i: (0, 0))


def _mean(a_ref, c_ref):
  cnt = jnp.maximum(c_ref[0, :, 0:1] + c_ref[1, :, 0:1], 1.0)
  return (a_ref[0] + a_ref[1]) / cnt


def _dense2_body(aA, cA, aB, cB, x, wnA, wrA, bA, wnB, wrB, bB, o):
  h = (jnp.dot(_mean(aA, cA), wnA[...], preferred_element_type=F32)
       + jnp.dot(_mean(aB, cB), wnB[...], preferred_element_type=F32)
       + jnp.dot(x[...], wrA[...] + wrB[...], preferred_element_type=F32)
       + bA[...] + bB[...])
  o[...] = jnp.where(h >= 0, h, 0.01 * h)


_DENSE2 = pl.pallas_call(
    _dense2_body,
    grid=(N_PAD // BLK,),
    in_specs=[_acc_spec(), _cnt_spec(), _acc_spec(), _cnt_spec(), _x_spec(),
              _w_spec(), _w_spec(), _b_spec(), _w_spec(), _w_spec(), _b_spec()],
    out_specs=pl.BlockSpec((BLK, DIM), lambda i: (i, 0)),
    out_shape=jax.ShapeDtypeStruct((N_PAD, DIM), F32),
)


def _dense1_body(aA, cA, x, wn, wr, b, o):
  h = (jnp.dot(_mean(aA, cA), wn[...], preferred_element_type=F32)
       + jnp.dot(x[...], wr[...], preferred_element_type=F32)
       + b[...])
  o[...] = jnp.where(h >= 0, h, 0.01 * h)


_DENSE1 = pl.pallas_call(
    _dense1_body,
    grid=(N_PAD // BLK,),
    in_specs=[_acc_spec(), _cnt_spec(), _x_spec(),
              _w_spec(), _w_spec(), _b_spec()],
    out_specs=pl.BlockSpec((BLK, DIM), lambda i: (i, 0)),
    out_shape=jax.ShapeDtypeStruct((N_PAD, DIM), F32),
)


def _dense2_final_body(aA, cA, aB, cB, x, wnA, wrA, bA, wnB, wrB, bB,
                       wl, bl, o):
  h = (jnp.dot(_mean(aA, cA), wnA[...], preferred_element_type=F32)
       + jnp.dot(_mean(aB, cB), wnB[...], preferred_element_type=F32)
       + jnp.dot(x[...], wrA[...] + wrB[...], preferred_element_type=F32)
       + bA[...] + bB[...])
  xp2 = jnp.where(h >= 0, h, 0.01 * h)
  o[...] = jnp.dot(xp2, wl[...], preferred_element_type=F32) + bl[...]


_DENSE2_FINAL = pl.pallas_call(
    _dense2_final_body,
    grid=(N_PAD // BLK,),
    in_specs=[_acc_spec(), _cnt_spec(), _acc_spec(), _cnt_spec(), _x_spec(),
              _w_spec(), _w_spec(), _b_spec(), _w_spec(), _w_spec(), _b_spec(),
              pl.BlockSpec((DIM, DIM_OUT), lambda i: (0, 0)),
              pl.BlockSpec((1, DIM_OUT), lambda i: (0, 0))],
    out_specs=pl.BlockSpec((BLK, DIM_OUT), lambda i: (i, 0)),
    out_shape=jax.ShapeDtypeStruct((N_PAD, DIM_OUT), F32),
)


def _prep_edges(e):
  src = jnp.concatenate([e[0], jnp.zeros((E_PAD - NUM_E,), jnp.int32)])
  dst = jnp.concatenate(
      [e[1], jnp.full((E_PAD - NUM_E,), PAD_ROW, jnp.int32)])
  return src.reshape(NW, NCH, K), dst.reshape(NW, NCH, K)


def kernel(x_paper, x_author, edge_index_pcp, edge_index_awp, edge_index_pwa,
           Wn_0_pcp, Wr_0_pcp, bn_0_pcp, Wn_0_awp, Wr_0_awp, bn_0_awp,
           Wn_0_pwa, Wr_0_pwa, bn_0_pwa,
           Wn_1_pcp, Wr_1_pcp, bn_1_pcp, Wn_1_awp, Wr_1_awp, bn_1_awp,
           Wn_1_pwa, Wr_1_pwa, bn_1_pwa, W_lin, b_lin):
  sp, dp = _prep_edges(edge_index_pcp)
  sw, dw = _prep_edges(edge_index_awp)
  sa, da = _prep_edges(edge_index_pwa)
  xp = jnp.pad(x_paper, ((0, N_PAD - NUM_P), (0, 0)))
  xa = jnp.pad(x_author, ((0, N_PAD - NUM_A), (0, 0)))
  zeros_b = jnp.zeros((ROWS_PT, DIM), F32)
  ones_b = jnp.ones((K, DIM), F32)

  cnt_pcp, cnt_awp, cnt_pwa = _COUNTS(dp, dw, da, ones_b, zeros_b)
  a_pcp, a_awp, a_pwa = _SEG3(xp, xa, xp, sp, sw, sa, dp, dw, da, zeros_b)
  xp1 = _DENSE2(a_pcp, cnt_pcp, a_awp, cnt_awp, xp,
                Wn_0_pcp, Wr_0_pcp, bn_0_pcp.reshape(1, DIM),
                Wn_0_awp, Wr_0_awp, bn_0_awp.reshape(1, DIM))
  xa1 = _DENSE1(a_pwa, cnt_pwa, xa,
                Wn_0_pwa, Wr_0_pwa, bn_0_pwa.reshape(1, DIM))
  b_pcp, b_awp = _SEG2(xp1, xa1, sp, sw, dp, dw, zeros_b)
  out = _DENSE2_FINAL(b_pcp, cnt_pcp, b_awp, cnt_awp, xp1,
                      Wn_1_pcp, Wr_1_pcp, bn_1_pcp.reshape(1, DIM),
                      Wn_1_awp, Wr_1_awp, bn_1_awp.reshape(1, DIM),
                      W_lin, b_lin.reshape(1, DIM_OUT))
  return out[:NUM_P]


# speedup vs baseline: 2.3870x; 1.0769x over previous
import functools

import jax
import jax.numpy as jnp
from jax import lax
from jax.experimental import pallas as pl
from jax.experimental.pallas import tpu as pltpu
from jax.experimental.pallas import tpu_sc as plsc

F32 = jnp.float32

NUM_P = 10000
NUM_A = 10000
NUM_E = 200000
DIM = 128
DIM_OUT = 64

NC = 2
NS = 16
NW = NC * NS
K = 128
NCH = 50
EPW = NCH * K
E_PAD = NW * EPW
N_PAD = 10240
ROWS_PT = N_PAD // NS
PAD_ROW = 10000

_MESH = plsc.VectorSubcoreMesh(
    core_axis_name="c", subcore_axis_name="s", num_cores=NC, num_subcores=NS)


def _make_segsum(num_types):
  out_t = [jax.ShapeDtypeStruct((NC, N_PAD, DIM), F32)] * num_types
  NBUF = 2
  scratch = [
      pltpu.VMEM((NCH, K), jnp.int32),
      pltpu.VMEM((NCH, K), jnp.int32),
      pltpu.VMEM((NBUF, K, DIM), F32),
      pltpu.VMEM_SHARED((N_PAD, DIM), F32),
      pltpu.SemaphoreType.DMA,
  ]

  @functools.partial(pl.kernel, out_type=out_t, mesh=_MESH,
                     scratch_types=scratch)
  def seg_kernel(*refs):
    xs = refs[:num_types]
    srcs = refs[num_types:2 * num_types]
    dsts = refs[2 * num_types:3 * num_types]
    zeros_hbm = refs[3 * num_types]
    outs = refs[3 * num_types + 1:4 * num_types + 1]
    src_v, dst_v, rows_v, acc, gsem = refs[4 * num_types + 1:]
    c = lax.axis_index("c")
    s = lax.axis_index("s")
    wid = c * NS + s
    for t in range(num_types):
      pltpu.sync_copy(zeros_hbm, acc.at[pl.ds(s * ROWS_PT, ROWS_PT)])
      pltpu.sync_copy(srcs[t].at[wid], src_v)
      pltpu.sync_copy(dsts[t].at[wid], dst_v)
      plsc.subcore_barrier()

      for j in range(NBUF - 1):
        pltpu.async_copy(xs[t].at[src_v.at[j]], rows_v.at[j], gsem)

      def chunk(j, carry, t=t):
        cur = lax.rem(j, NBUF)
        pltpu.make_async_copy(xs[t].at[src_v.at[j]], rows_v.at[cur],
                              gsem).wait()
        nxt_j = j + (NBUF - 1)

        @pl.when(nxt_j < NCH)
        def _():
          pltpu.async_copy(xs[t].at[src_v.at[nxt_j]],
                           rows_v.at[lax.rem(nxt_j, NBUF)], gsem)

        pltpu.sync_copy(rows_v.at[cur], acc.at[dst_v.at[j]], add=True)
        return carry

      lax.fori_loop(0, NCH, chunk, 0)
      plsc.subcore_barrier()
      pltpu.sync_copy(acc.at[pl.ds(s * ROWS_PT, ROWS_PT)],
                      outs[t].at[c, pl.ds(s * ROWS_PT, ROWS_PT)])

  return seg_kernel


def _make_counts():
  out_t = [jax.ShapeDtypeStruct((NC, N_PAD, DIM), F32)] * 3
  scratch = [
      pltpu.VMEM((NCH, K), jnp.int32),
      pltpu.VMEM((K, DIM), F32),
      pltpu.VMEM_SHARED((N_PAD, DIM), F32),
  ]

  @functools.partial(pl.kernel, out_type=out_t, mesh=_MESH,
                     scratch_types=scratch)
  def cnt_kernel(d0, d1, d2, ones_hbm, zeros_hbm, o0, o1, o2,
                 dst_v, ones_v, acc):
    c = lax.axis_index("c")
    s = lax.axis_index("s")
    wid = c * NS + s
    pltpu.sync_copy(ones_hbm, ones_v)
    for t, (dref, oref) in enumerate(((d0, o0), (d1, o1), (d2, o2))):
      pltpu.sync_copy(zeros_hbm, acc.at[pl.ds(s * ROWS_PT, ROWS_PT)])
      pltpu.sync_copy(dref.at[wid], dst_v)
      plsc.subcore_barrier()

      def chunk(j, carry, t=t):
        pltpu.sync_copy(ones_v, acc.at[dst_v.at[j]], add=True)
        return carry

      lax.fori_loop(0, NCH, chunk, 0)
      plsc.subcore_barrier()
      pltpu.sync_copy(acc.at[pl.ds(s * ROWS_PT, ROWS_PT)],
                      oref.at[c, pl.ds(s * ROWS_PT, ROWS_PT)])

  return cnt_kernel


_SEG3 = _make_segsum(3)
_SEG2 = _make_segsum(2)
_COUNTS = _make_counts()

BLK = 2048


def _acc_spec():
  return pl.BlockSpec((NC, BLK, DIM), lambda i: (0, i, 0))


def _cnt_spec():
  return pl.BlockSpec((NC, BLK, DIM), lambda i: (0, i, 0))


def _x_spec():
  return pl.BlockSpec((BLK, DIM), lambda i: (i, 0))


def _w_spec():
  return pl.BlockSpec((DIM, DIM), lambda i: (0, 0))


def _b_spec():
  return pl.BlockSpec((1, DIM), lambda i: (0, 0))


def _mean(a_ref, c_ref):
  cnt = jnp.maximum(c_ref[0, :, 0:1] + c_ref[1, :, 0:1], 1.0)
  return (a_ref[0] + a_ref[1]) / cnt


def _dense2_body(aA, cA, aB, cB, x, wnA, wrA, bA, wnB, wrB, bB, o):
  h = (jnp.dot(_mean(aA, cA), wnA[...], preferred_element_type=F32)
       + jnp.dot(_mean(aB, cB), wnB[...], preferred_element_type=F32)
       + jnp.dot(x[...], wrA[...] + wrB[...], preferred_element_type=F32)
       + bA[...] + bB[...])
  o[...] = jnp.where(h >= 0, h, 0.01 * h)


_DENSE2 = pl.pallas_call(
    _dense2_body,
    grid=(N_PAD // BLK,),
    in_specs=[_acc_spec(), _cnt_spec(), _acc_spec(), _cnt_spec(), _x_spec(),
              _w_spec(), _w_spec(), _b_spec(), _w_spec(), _w_spec(), _b_spec()],
    out_specs=pl.BlockSpec((BLK, DIM), lambda i: (i, 0)),
    out_shape=jax.ShapeDtypeStruct((N_PAD, DIM), F32),
)


def _dense1_body(aA, cA, x, wn, wr, b, o):
  h = (jnp.dot(_mean(aA, cA), wn[...], preferred_element_type=F32)
       + jnp.dot(x[...], wr[...], preferred_element_type=F32)
       + b[...])
  o[...] = jnp.where(h >= 0, h, 0.01 * h)


_DENSE1 = pl.pallas_call(
    _dense1_body,
    grid=(N_PAD // BLK,),
    in_specs=[_acc_spec(), _cnt_spec(), _x_spec(),
              _w_spec(), _w_spec(), _b_spec()],
    out_specs=pl.BlockSpec((BLK, DIM), lambda i: (i, 0)),
    out_shape=jax.ShapeDtypeStruct((N_PAD, DIM), F32),
)


def _dense2_final_body(aA, cA, aB, cB, x, wnA, wrA, bA, wnB, wrB, bB,
                       wl, bl, o):
  h = (jnp.dot(_mean(aA, cA), wnA[...], preferred_element_type=F32)
       + jnp.dot(_mean(aB, cB), wnB[...], preferred_element_type=F32)
       + jnp.dot(x[...], wrA[...] + wrB[...], preferred_element_type=F32)
       + bA[...] + bB[...])
  xp2 = jnp.where(h >= 0, h, 0.01 * h)
  o[...] = jnp.dot(xp2, wl[...], preferred_element_type=F32) + bl[...]


_DENSE2_FINAL = pl.pallas_call(
    _dense2_final_body,
    grid=(N_PAD // BLK,),
    in_specs=[_acc_spec(), _cnt_spec(), _acc_spec(), _cnt_spec(), _x_spec(),
              _w_spec(), _w_spec(), _b_spec(), _w_spec(), _w_spec(), _b_spec(),
              pl.BlockSpec((DIM, DIM_OUT), lambda i: (0, 0)),
              pl.BlockSpec((1, DIM_OUT), lambda i: (0, 0))],
    out_specs=pl.BlockSpec((BLK, DIM_OUT), lambda i: (i, 0)),
    out_shape=jax.ShapeDtypeStruct((N_PAD, DIM_OUT), F32),
)


def _prep_edges(e):
  src = jnp.concatenate([e[0], jnp.zeros((E_PAD - NUM_E,), jnp.int32)])
  dst = jnp.concatenate(
      [e[1], jnp.full((E_PAD - NUM_E,), PAD_ROW, jnp.int32)])
  return src.reshape(NW, NCH, K), dst.reshape(NW, NCH, K)


def kernel(x_paper, x_author, edge_index_pcp, edge_index_awp, edge_index_pwa,
           Wn_0_pcp, Wr_0_pcp, bn_0_pcp, Wn_0_awp, Wr_0_awp, bn_0_awp,
           Wn_0_pwa, Wr_0_pwa, bn_0_pwa,
           Wn_1_pcp, Wr_1_pcp, bn_1_pcp, Wn_1_awp, Wr_1_awp, bn_1_awp,
           Wn_1_pwa, Wr_1_pwa, bn_1_pwa, W_lin, b_lin):
  sp, dp = _prep_edges(edge_index_pcp)
  sw, dw = _prep_edges(edge_index_awp)
  sa, da = _prep_edges(edge_index_pwa)
  xp = jnp.pad(x_paper, ((0, N_PAD - NUM_P), (0, 0)))
  xa = jnp.pad(x_author, ((0, N_PAD - NUM_A), (0, 0)))
  zeros_b = jnp.zeros((ROWS_PT, DIM), F32)
  ones_b = jnp.ones((K, DIM), F32)

  cnt_pcp, cnt_awp, cnt_pwa = _COUNTS(dp, dw, da, ones_b, zeros_b)
  a_pcp, a_awp, a_pwa = _SEG3(xp, xa, xp, sp, sw, sa, dp, dw, da, zeros_b)
  xp1 = _DENSE2(a_pcp, cnt_pcp, a_awp, cnt_awp, xp,
                Wn_0_pcp, Wr_0_pcp, bn_0_pcp.reshape(1, DIM),
                Wn_0_awp, Wr_0_awp, bn_0_awp.reshape(1, DIM))
  xa1 = _DENSE1(a_pwa, cnt_pwa, xa,
                Wn_0_pwa, Wr_0_pwa, bn_0_pwa.reshape(1, DIM))
  b_pcp, b_awp = _SEG2(xp1, xa1, sp, sw, dp, dw, zeros_b)
  out = _DENSE2_FINAL(b_pcp, cnt_pcp, b_awp, cnt_awp, xp1,
                      Wn_1_pcp, Wr_1_pcp, bn_1_pcp.reshape(1, DIM),
                      Wn_1_awp, Wr_1_awp, bn_1_awp.reshape(1, DIM),
                      W_lin, b_lin.reshape(1, DIM_OUT))
  return out[:NUM_P]
